# Initial kernel scaffold; baseline (speedup 1.0000x reference)
import jax
import jax.numpy as jnp
from jax.experimental import pallas as pl


def kernel(x, edge_idx, table):
    raise NotImplementedError("write your pallas kernel here")


# speedup vs baseline: 13.2472x; 13.2472x over previous
import functools

import jax
import jax.numpy as jnp
from jax import lax
from jax.experimental import pallas as pl
from jax.experimental.pallas import tpu as pltpu
from jax.experimental.pallas import tpu_sc as plsc

N = 1024
E = 16384
HEADS = 16
TABLE_V = 50
MAX_ITERS = 49

NC = 2
NS = 16
LANES = 16
NW = NC * NS

ROWS_PER_TILE = N // NW
IDX_PER_TILE = (N * N) // NW
CHUNK = 4096

_mesh = plsc.VectorSubcoreMesh(core_axis_name="c", subcore_axis_name="s")


@functools.partial(
    pl.kernel,
    out_type=jax.ShapeDtypeStruct((N * N,), jnp.float32),
    mesh=_mesh,
    scratch_types=[
        pltpu.VMEM((2, E), jnp.int32),
        pltpu.VMEM((ROWS_PER_TILE * N,), jnp.float32),
    ],
    compiler_params=pltpu.CompilerParams(needs_layout_passes=False),
)
def _build_adj(edge_hbm, a_hbm, edges_v, a_v):
    wid = lax.axis_index("s") * NC + lax.axis_index("c")
    base_row = wid * ROWS_PER_TILE
    pltpu.sync_copy(edge_hbm, edges_v)

    def zero_body(i, carry):
        a_v[pl.ds(i * LANES, LANES)] = jnp.zeros((LANES,), jnp.float32)
        return carry

    lax.fori_loop(0, ROWS_PER_TILE * N // LANES, zero_body, 0)

    ones = jnp.ones((LANES,), jnp.float32)

    def edge_body(i, carry):
        src = edges_v[0, pl.ds(i * LANES, LANES)]
        dst = edges_v[1, pl.ds(i * LANES, LANES)]
        mask = (src >= base_row) & (src < base_row + ROWS_PER_TILE)
        local = jnp.where(mask, (src - base_row) * N + dst, 0)
        plsc.store_scatter(a_v, [local], ones, mask=mask)
        return carry

    lax.fori_loop(0, E // LANES, edge_body, 0)
    pltpu.sync_copy(a_v, a_hbm.at[pl.ds(base_row * N, ROWS_PER_TILE * N)])


def _bfs_body(a_ref, dist_ref, abf_ref, frontier_ref, cnt_ref):
    abf_ref[...] = a_ref[...].astype(jnp.bfloat16)
    row = lax.broadcasted_iota(jnp.int32, (N, N), 0)
    col = lax.broadcasted_iota(jnp.int32, (N, N), 1)
    diag = row == col
    frontier_ref[...] = jnp.where(diag, 1.0, 0.0).astype(jnp.bfloat16)
    dist_ref[...] = jnp.zeros((N, N), jnp.int32)

    def body(i, done):
        @pl.when(jnp.logical_not(done))
        def _():
            nxt = lax.dot_general(
                frontier_ref[...], abf_ref[...],
                (((1,), (0,)), ((), ())),
                preferred_element_type=jnp.float32,
            )
            new = (nxt > 0.0) & (dist_ref[...] == 0) & jnp.logical_not(diag)
            dist_ref[...] = jnp.where(new, i + 1, dist_ref[...])
            frontier_ref[...] = new.astype(jnp.bfloat16)
            cnt_ref[0] = jnp.sum(new.astype(jnp.int32))

        return jnp.logical_or(done, cnt_ref[0] == 0)

    lax.fori_loop(0, MAX_ITERS, body, False)


_bfs = pl.pallas_call(
    _bfs_body,
    out_shape=jax.ShapeDtypeStruct((N, N), jnp.int32),
    scratch_shapes=[
        pltpu.VMEM((N, N), jnp.bfloat16),
        pltpu.VMEM((N, N), jnp.bfloat16),
        pltpu.SMEM((1,), jnp.int32),
    ],
)


@functools.partial(
    pl.kernel,
    out_type=jax.ShapeDtypeStruct((HEADS, N * N), jnp.float32),
    mesh=_mesh,
    scratch_types=[
        pltpu.VMEM((HEADS * TABLE_V,), jnp.float32),
        pltpu.VMEM((CHUNK,), jnp.int32),
        pltpu.VMEM((HEADS, CHUNK), jnp.float32),
    ],
    compiler_params=pltpu.CompilerParams(needs_layout_passes=False),
)
def _lookup(dist_hbm, ftab_hbm, out_hbm, tab_v, idx_v, out_v):
    wid = lax.axis_index("s") * NC + lax.axis_index("c")
    base = wid * IDX_PER_TILE
    pltpu.sync_copy(ftab_hbm, tab_v)

    def chunk_body(cidx, carry):
        off = base + cidx * CHUNK
        pltpu.sync_copy(dist_hbm.at[pl.ds(off, CHUNK)], idx_v)

        def g_body(g, inner):
            iv = idx_v[pl.ds(g * LANES, LANES)]
            for h in range(HEADS):
                vals = plsc.load_gather(tab_v, [iv + h * TABLE_V])
                out_v[h, pl.ds(g * LANES, LANES)] = vals
            return inner

        lax.fori_loop(0, CHUNK // LANES, g_body, 0)
        for h in range(HEADS):
            pltpu.sync_copy(out_v.at[h], out_hbm.at[h, pl.ds(off, CHUNK)])
        return carry

    lax.fori_loop(0, IDX_PER_TILE // CHUNK, chunk_body, 0)


def kernel(x, edge_idx, table):
    del x
    a_flat = _build_adj(edge_idx)
    dist = _bfs(a_flat.reshape(N, N))
    ftab = jnp.transpose(table).reshape(HEADS * TABLE_V)
    out = _lookup(dist.reshape(N * N), ftab)
    return out.reshape(HEADS, N, N)


# speedup vs baseline: 19.2828x; 1.4556x over previous
import functools

import jax
import jax.numpy as jnp
from jax import lax
from jax.experimental import pallas as pl
from jax.experimental.pallas import tpu as pltpu
from jax.experimental.pallas import tpu_sc as plsc

N = 1024
E = 16384
HEADS = 16
TABLE_V = 50
MAX_ITERS = 49

NC = 2
NS = 16
LANES = 16
NW = NC * NS

ROWS_PER_TILE = N // NW
CHUNK = 2048
ROWS_PER_CHUNK = CHUNK // N
CHUNKS = ROWS_PER_TILE // ROWS_PER_CHUNK

_mesh = plsc.VectorSubcoreMesh(core_axis_name="c", subcore_axis_name="s")
_sc_params = pltpu.CompilerParams(needs_layout_passes=False)


@functools.partial(
    pl.kernel,
    out_type=jax.ShapeDtypeStruct((N, N), jnp.float32),
    mesh=_mesh,
    scratch_types=[
        pltpu.VMEM((2, E), jnp.int32),
        pltpu.VMEM((ROWS_PER_TILE, N), jnp.float32),
    ],
    compiler_params=_sc_params,
)
def _build_adj(edge_hbm, a_hbm, edges_v, a_v):
    wid = lax.axis_index("s") * NC + lax.axis_index("c")
    base_row = wid * ROWS_PER_TILE
    pltpu.sync_copy(edge_hbm, edges_v)

    def zero_body(i, carry):
        r = lax.shift_right_logical(i, 6)
        c = lax.shift_left(jnp.bitwise_and(i, 63), 4)
        a_v[r, pl.ds(c, LANES)] = jnp.zeros((LANES,), jnp.float32)
        return carry

    lax.fori_loop(0, ROWS_PER_TILE * N // LANES, zero_body, 0)

    ones = jnp.ones((LANES,), jnp.float32)

    def edge_body(i, carry):
        src = edges_v[0, pl.ds(i * LANES, LANES)]
        dst = edges_v[1, pl.ds(i * LANES, LANES)]
        mask = (src >= base_row) & (src < base_row + ROWS_PER_TILE)
        local_r = jnp.where(mask, src - base_row, 0)
        plsc.store_scatter(a_v, [local_r, dst], ones, mask=mask)
        return carry

    lax.fori_loop(0, E // LANES, edge_body, 0)
    pltpu.sync_copy(a_v, a_hbm.at[pl.ds(base_row, ROWS_PER_TILE), :])


def _bfs_body(a_ref, dist_ref, abf_ref, frontier_ref, cnt_ref):
    abf_ref[...] = a_ref[...].astype(jnp.bfloat16)
    row = lax.broadcasted_iota(jnp.int32, (N, N), 0)
    col = lax.broadcasted_iota(jnp.int32, (N, N), 1)
    diag = row == col
    frontier_ref[...] = jnp.where(diag, 1.0, 0.0).astype(jnp.bfloat16)
    dist_ref[...] = jnp.zeros((N, N), jnp.int32)

    def body(i, done):
        @pl.when(jnp.logical_not(done))
        def _():
            nxt = lax.dot_general(
                frontier_ref[...], abf_ref[...],
                (((1,), (0,)), ((), ())),
                preferred_element_type=jnp.float32,
            )
            new = (nxt > 0.0) & (dist_ref[...] == 0) & jnp.logical_not(diag)
            dist_ref[...] = jnp.where(new, i + 1, dist_ref[...])
            frontier_ref[...] = new.astype(jnp.bfloat16)
            cnt_ref[0] = jnp.sum(new.astype(jnp.int32))

        return jnp.logical_or(done, cnt_ref[0] == 0)

    lax.fori_loop(0, MAX_ITERS, body, False)


_bfs = pl.pallas_call(
    _bfs_body,
    out_shape=jax.ShapeDtypeStruct((N, N), jnp.int32),
    scratch_shapes=[
        pltpu.VMEM((N, N), jnp.bfloat16),
        pltpu.VMEM((N, N), jnp.bfloat16),
        pltpu.SMEM((1,), jnp.int32),
    ],
)


@functools.partial(
    pl.kernel,
    out_type=jax.ShapeDtypeStruct((HEADS, N, N), jnp.float32),
    mesh=_mesh,
    scratch_types=[
        pltpu.VMEM((HEADS * TABLE_V,), jnp.float32),
        pltpu.VMEM((2, ROWS_PER_CHUNK, N), jnp.int32),
        pltpu.VMEM((2, HEADS, ROWS_PER_CHUNK, N), jnp.float32),
        pltpu.SemaphoreType.DMA((2,)),
        pltpu.SemaphoreType.DMA((2,)),
    ],
    compiler_params=_sc_params,
)
def _lookup(dist_hbm, ftab_hbm, out_hbm, tab_v, idx_v, out_v, in_sems, out_sems):
    wid = lax.axis_index("s") * NC + lax.axis_index("c")
    row_base = wid * ROWS_PER_TILE
    pltpu.sync_copy(ftab_hbm, tab_v)

    def start_in(c):
        return pltpu.async_copy(
            dist_hbm.at[pl.ds(row_base + c * ROWS_PER_CHUNK, ROWS_PER_CHUNK), :],
            idx_v.at[c % 2],
            in_sems.at[c % 2],
        )

    in_cp = [start_in(0), None]
    out_cp = [None, None]
    for c in range(CHUNKS):
        b = c % 2
        if c + 1 < CHUNKS:
            in_cp[(c + 1) % 2] = start_in(c + 1)
        in_cp[b].wait()
        if out_cp[b] is not None:
            out_cp[b].wait()
        for r in range(ROWS_PER_CHUNK):
            def cg_body(cg, carry):
                col = cg * LANES
                iv = idx_v[b, r, pl.ds(col, LANES)]
                for h in range(HEADS):
                    vals = plsc.load_gather(tab_v, [iv + h * TABLE_V])
                    out_v[b, h, r, pl.ds(col, LANES)] = vals
                return carry

            lax.fori_loop(0, N // LANES, cg_body, 0)
        out_cp[b] = pltpu.async_copy(
            out_v.at[b],
            out_hbm.at[:, pl.ds(row_base + c * ROWS_PER_CHUNK, ROWS_PER_CHUNK), :],
            out_sems.at[b],
        )
    out_cp[(CHUNKS - 2) % 2].wait()
    out_cp[(CHUNKS - 1) % 2].wait()


def kernel(x, edge_idx, table):
    del x
    a = _build_adj(edge_idx)
    dist = _bfs(a)
    ftab = jnp.transpose(table).reshape(HEADS * TABLE_V)
    return _lookup(dist, ftab)


# speedup vs baseline: 38.0661x; 1.9741x over previous
import functools

import jax
import jax.numpy as jnp
from jax import lax
from jax.experimental import pallas as pl
from jax.experimental.pallas import tpu as pltpu
from jax.experimental.pallas import tpu_sc as plsc

N = 1024
E = 16384
HEADS = 16
TABLE_V = 50
MAX_ITERS = 49

NC = 2
NS = 16
LANES = 16
NW = NC * NS

ROWS_PER_TILE = N // NW
CHUNK = 2048
ROWS_PER_CHUNK = CHUNK // N
CHUNKS = ROWS_PER_TILE // ROWS_PER_CHUNK

_mesh = plsc.VectorSubcoreMesh(core_axis_name="c", subcore_axis_name="s")
_sc_params = pltpu.CompilerParams(needs_layout_passes=False)


@functools.partial(
    pl.kernel,
    out_type=jax.ShapeDtypeStruct((N, N), jnp.float32),
    mesh=_mesh,
    scratch_types=[
        pltpu.VMEM((2, E), jnp.int32),
        pltpu.VMEM((ROWS_PER_TILE, N), jnp.float32),
    ],
    compiler_params=_sc_params,
)
def _build_adj(edge_hbm, a_hbm, edges_v, a_v):
    wid = lax.axis_index("s") * NC + lax.axis_index("c")
    base_row = wid * ROWS_PER_TILE
    pltpu.sync_copy(edge_hbm, edges_v)

    def zero_body(i, carry):
        r = lax.shift_right_logical(i, 6)
        c = lax.shift_left(jnp.bitwise_and(i, 63), 4)
        a_v[r, pl.ds(c, LANES)] = jnp.zeros((LANES,), jnp.float32)
        return carry

    lax.fori_loop(0, ROWS_PER_TILE * N // LANES, zero_body, 0)

    ones = jnp.ones((LANES,), jnp.float32)

    def edge_body(i, carry):
        src = edges_v[0, pl.ds(i * LANES, LANES)]
        dst = edges_v[1, pl.ds(i * LANES, LANES)]
        mask = (src >= base_row) & (src < base_row + ROWS_PER_TILE)
        local_r = jnp.where(mask, src - base_row, 0)
        plsc.store_scatter(a_v, [local_r, dst], ones, mask=mask)
        return carry

    lax.fori_loop(0, E // LANES, edge_body, 0)
    pltpu.sync_copy(a_v, a_hbm.at[pl.ds(base_row, ROWS_PER_TILE), :])


def _bfs_body(a_ref, dist_ref, abf_ref, frontier_ref, cnt_ref):
    abf_ref[...] = a_ref[...].astype(jnp.bfloat16)
    row = lax.broadcasted_iota(jnp.int32, (N, N), 0)
    col = lax.broadcasted_iota(jnp.int32, (N, N), 1)
    diag = row == col
    frontier_ref[...] = jnp.where(diag, 1.0, 0.0).astype(jnp.bfloat16)
    dist_ref[...] = jnp.zeros((N, N), jnp.int32)

    def body(i, done):
        @pl.when(jnp.logical_not(done))
        def _():
            nxt = lax.dot_general(
                frontier_ref[...], abf_ref[...],
                (((1,), (0,)), ((), ())),
                preferred_element_type=jnp.float32,
            )
            new = (nxt > 0.0) & (dist_ref[...] == 0) & jnp.logical_not(diag)
            dist_ref[...] = jnp.where(new, i + 1, dist_ref[...])
            frontier_ref[...] = new.astype(jnp.bfloat16)
            cnt_ref[0] = jnp.sum(new.astype(jnp.int32))

        return jnp.logical_or(done, cnt_ref[0] == 0)

    lax.fori_loop(0, MAX_ITERS, body, False)


_bfs = pl.pallas_call(
    _bfs_body,
    out_shape=jax.ShapeDtypeStruct((N, N), jnp.int32),
    scratch_shapes=[
        pltpu.VMEM((N, N), jnp.bfloat16),
        pltpu.VMEM((N, N), jnp.bfloat16),
        pltpu.SMEM((1,), jnp.int32),
    ],
)


@functools.partial(
    pl.kernel,
    out_type=jax.ShapeDtypeStruct((HEADS, N, N), jnp.float32),
    mesh=_mesh,
    scratch_types=[
        pltpu.VMEM((HEADS * TABLE_V,), jnp.float32),
        pltpu.VMEM((2, ROWS_PER_CHUNK, N), jnp.int32),
        pltpu.VMEM((2, HEADS, ROWS_PER_CHUNK, N), jnp.float32),
        pltpu.SemaphoreType.DMA((2,)),
        pltpu.SemaphoreType.DMA((2,)),
    ],
    compiler_params=_sc_params,
)
def _lookup(dist_hbm, ftab_hbm, out_hbm, tab_v, idx_v, out_v, in_sems, out_sems):
    wid = lax.axis_index("s") * NC + lax.axis_index("c")
    row_base = wid * ROWS_PER_TILE
    pltpu.sync_copy(ftab_hbm, tab_v)

    def start_in(c):
        return pltpu.async_copy(
            dist_hbm.at[pl.ds(row_base + c * ROWS_PER_CHUNK, ROWS_PER_CHUNK), :],
            idx_v.at[c % 2],
            in_sems.at[c % 2],
        )

    in_cp = [start_in(0), None]
    out_cp = [None, None]
    for c in range(CHUNKS):
        b = c % 2
        if c + 1 < CHUNKS:
            in_cp[(c + 1) % 2] = start_in(c + 1)
        in_cp[b].wait()
        if out_cp[b] is not None:
            out_cp[b].wait()
        for r in range(ROWS_PER_CHUNK):
            @plsc.parallel_loop(0, N // LANES, 1, unroll=2)
            def _(cg):
                col = cg * LANES
                iv = idx_v[b, r, pl.ds(col, LANES)]
                for h in range(HEADS):
                    vals = plsc.load_gather(tab_v, [iv + h * TABLE_V])
                    out_v[b, h, r, pl.ds(col, LANES)] = vals
        out_cp[b] = pltpu.async_copy(
            out_v.at[b],
            out_hbm.at[:, pl.ds(row_base + c * ROWS_PER_CHUNK, ROWS_PER_CHUNK), :],
            out_sems.at[b],
        )
    out_cp[(CHUNKS - 2) % 2].wait()
    out_cp[(CHUNKS - 1) % 2].wait()


def kernel(x, edge_idx, table):
    del x
    a = _build_adj(edge_idx)
    dist = _bfs(a)
    ftab = jnp.transpose(table).reshape(HEADS * TABLE_V)
    return _lookup(dist, ftab)


# speedup vs baseline: 43.5136x; 1.1431x over previous
import functools

import jax
import jax.numpy as jnp
from jax import lax
from jax.experimental import pallas as pl
from jax.experimental.pallas import tpu as pltpu
from jax.experimental.pallas import tpu_sc as plsc

N = 1024
E = 16384
HEADS = 16
TABLE_V = 50
MAX_ITERS = 49

NC = 2
NS = 16
LANES = 16
NW = NC * NS

ROWS_PER_TILE = N // NW
CHUNK = 2048
ROWS_PER_CHUNK = CHUNK // N
CHUNKS = ROWS_PER_TILE // ROWS_PER_CHUNK

_mesh = plsc.VectorSubcoreMesh(core_axis_name="c", subcore_axis_name="s")
_sc_params = pltpu.CompilerParams(needs_layout_passes=False)


@functools.partial(
    pl.kernel,
    out_type=jax.ShapeDtypeStruct((N, N), jnp.float32),
    mesh=_mesh,
    scratch_types=[
        pltpu.VMEM((2, E), jnp.int32),
        pltpu.VMEM((ROWS_PER_TILE, N), jnp.float32),
    ],
    compiler_params=_sc_params,
)
def _build_adj(edge_hbm, a_hbm, edges_v, a_v):
    wid = lax.axis_index("s") * NC + lax.axis_index("c")
    base_row = wid * ROWS_PER_TILE
    pltpu.sync_copy(edge_hbm, edges_v)

    @plsc.parallel_loop(0, ROWS_PER_TILE * N // LANES, 1, unroll=4)
    def _(i):
        r = lax.shift_right_logical(i, 6)
        c = lax.shift_left(jnp.bitwise_and(i, 63), 4)
        a_v[r, pl.ds(c, LANES)] = jnp.zeros((LANES,), jnp.float32)

    ones = jnp.ones((LANES,), jnp.float32)

    @plsc.parallel_loop(0, E // LANES, 1, unroll=4)
    def _(i):
        src = edges_v[0, pl.ds(i * LANES, LANES)]
        dst = edges_v[1, pl.ds(i * LANES, LANES)]
        mask = (src >= base_row) & (src < base_row + ROWS_PER_TILE)
        local_r = jnp.where(mask, src - base_row, 0)
        plsc.store_scatter(a_v, [local_r, dst], ones, mask=mask)
    pltpu.sync_copy(a_v, a_hbm.at[pl.ds(base_row, ROWS_PER_TILE), :])


def _bfs_body(a_ref, dist_ref, abf_ref, frontier_ref, cnt_ref):
    abf_ref[...] = a_ref[...].astype(jnp.bfloat16)
    row = lax.broadcasted_iota(jnp.int32, (N, N), 0)
    col = lax.broadcasted_iota(jnp.int32, (N, N), 1)
    diag = row == col
    frontier_ref[...] = jnp.where(diag, 1.0, 0.0).astype(jnp.bfloat16)
    dist_ref[...] = jnp.zeros((N, N), jnp.int32)

    def body(i, done):
        @pl.when(jnp.logical_not(done))
        def _():
            nxt = lax.dot_general(
                frontier_ref[...], abf_ref[...],
                (((1,), (0,)), ((), ())),
                preferred_element_type=jnp.float32,
            )
            new = (nxt > 0.0) & (dist_ref[...] == 0) & jnp.logical_not(diag)
            dist_ref[...] = jnp.where(new, i + 1, dist_ref[...])
            frontier_ref[...] = new.astype(jnp.bfloat16)
            cnt_ref[0] = jnp.sum(new.astype(jnp.int32))

        return jnp.logical_or(done, cnt_ref[0] == 0)

    lax.fori_loop(0, MAX_ITERS, body, False)


_bfs = pl.pallas_call(
    _bfs_body,
    out_shape=jax.ShapeDtypeStruct((N, N), jnp.int32),
    scratch_shapes=[
        pltpu.VMEM((N, N), jnp.bfloat16),
        pltpu.VMEM((N, N), jnp.bfloat16),
        pltpu.SMEM((1,), jnp.int32),
    ],
)


@functools.partial(
    pl.kernel,
    out_type=jax.ShapeDtypeStruct((HEADS, N, N), jnp.float32),
    mesh=_mesh,
    scratch_types=[
        pltpu.VMEM((HEADS * TABLE_V,), jnp.float32),
        pltpu.VMEM((2, ROWS_PER_CHUNK, N), jnp.int32),
        pltpu.VMEM((2, HEADS, ROWS_PER_CHUNK, N), jnp.float32),
        pltpu.SemaphoreType.DMA((2,)),
        pltpu.SemaphoreType.DMA((2,)),
    ],
    compiler_params=_sc_params,
)
def _lookup(dist_hbm, ftab_hbm, out_hbm, tab_v, idx_v, out_v, in_sems, out_sems):
    wid = lax.axis_index("s") * NC + lax.axis_index("c")
    row_base = wid * ROWS_PER_TILE
    pltpu.sync_copy(ftab_hbm, tab_v)

    def start_in(c):
        return pltpu.async_copy(
            dist_hbm.at[pl.ds(row_base + c * ROWS_PER_CHUNK, ROWS_PER_CHUNK), :],
            idx_v.at[c % 2],
            in_sems.at[c % 2],
        )

    in_cp = [start_in(0), None]
    out_cp = [None, None]
    for c in range(CHUNKS):
        b = c % 2
        if c + 1 < CHUNKS:
            in_cp[(c + 1) % 2] = start_in(c + 1)
        in_cp[b].wait()
        if out_cp[b] is not None:
            out_cp[b].wait()
        for r in range(ROWS_PER_CHUNK):
            @plsc.parallel_loop(0, N // LANES, 1, unroll=2)
            def _(cg):
                col = cg * LANES
                iv = idx_v[b, r, pl.ds(col, LANES)]
                for h in range(HEADS):
                    vals = plsc.load_gather(tab_v, [iv + h * TABLE_V])
                    out_v[b, h, r, pl.ds(col, LANES)] = vals
        out_cp[b] = pltpu.async_copy(
            out_v.at[b],
            out_hbm.at[:, pl.ds(row_base + c * ROWS_PER_CHUNK, ROWS_PER_CHUNK), :],
            out_sems.at[b],
        )
    out_cp[(CHUNKS - 2) % 2].wait()
    out_cp[(CHUNKS - 1) % 2].wait()


def kernel(x, edge_idx, table):
    del x
    a = _build_adj(edge_idx)
    dist = _bfs(a)
    ftab = jnp.transpose(table).reshape(HEADS * TABLE_V)
    return _lookup(dist, ftab)


# speedup vs baseline: 47.2926x; 1.0868x over previous
import functools

import jax
import jax.numpy as jnp
from jax import lax
from jax.experimental import pallas as pl
from jax.experimental.pallas import tpu as pltpu
from jax.experimental.pallas import tpu_sc as plsc

N = 1024
E = 16384
HEADS = 16
TABLE_V = 50
MAX_ITERS = 49

NC = 2
NS = 16
LANES = 16
NW = NC * NS

ROWS_PER_TILE = N // NW
CHUNK = 2048
ROWS_PER_CHUNK = CHUNK // N
CHUNKS = ROWS_PER_TILE // ROWS_PER_CHUNK

_mesh = plsc.VectorSubcoreMesh(core_axis_name="c", subcore_axis_name="s")
_sc_params = pltpu.CompilerParams(needs_layout_passes=False)


@functools.partial(
    pl.kernel,
    out_type=jax.ShapeDtypeStruct((N, N), jnp.float32),
    mesh=_mesh,
    scratch_types=[
        pltpu.VMEM((2, E), jnp.int32),
        pltpu.VMEM((ROWS_PER_TILE, N), jnp.float32),
    ],
    compiler_params=_sc_params,
)
def _build_adj(edge_hbm, a_hbm, edges_v, a_v):
    wid = lax.axis_index("s") * NC + lax.axis_index("c")
    base_row = wid * ROWS_PER_TILE
    pltpu.sync_copy(edge_hbm, edges_v)

    @plsc.parallel_loop(0, ROWS_PER_TILE * N // LANES, 1, unroll=4)
    def _(i):
        r = lax.shift_right_logical(i, 6)
        c = lax.shift_left(jnp.bitwise_and(i, 63), 4)
        a_v[r, pl.ds(c, LANES)] = jnp.zeros((LANES,), jnp.float32)

    ones = jnp.ones((LANES,), jnp.float32)

    @plsc.parallel_loop(0, E // LANES, 1, unroll=4)
    def _(i):
        src = edges_v[0, pl.ds(i * LANES, LANES)]
        dst = edges_v[1, pl.ds(i * LANES, LANES)]
        mask = (src >= base_row) & (src < base_row + ROWS_PER_TILE)
        local_r = jnp.where(mask, src - base_row, 0)
        plsc.store_scatter(a_v, [local_r, dst], ones, mask=mask)
    pltpu.sync_copy(a_v, a_hbm.at[pl.ds(base_row, ROWS_PER_TILE), :])


def _bfs_body(a_ref, dist_ref, abf_ref, frontier_ref, cnt_ref):
    abf_ref[...] = a_ref[...].astype(jnp.bfloat16)
    row = lax.broadcasted_iota(jnp.int32, (N, N), 0)
    col = lax.broadcasted_iota(jnp.int32, (N, N), 1)
    diag = row == col
    frontier_ref[...] = jnp.where(diag, 1.0, 0.0).astype(jnp.bfloat16)
    dist_ref[...] = jnp.zeros((N, N), jnp.int32)

    def body(i, done):
        @pl.when(jnp.logical_not(done))
        def _():
            nxt = lax.dot_general(
                frontier_ref[...], abf_ref[...],
                (((1,), (0,)), ((), ())),
                preferred_element_type=jnp.float32,
            )
            new = (nxt > 0.0) & (dist_ref[...] == 0) & jnp.logical_not(diag)
            dist_ref[...] = jnp.where(new, i + 1, dist_ref[...])
            frontier_ref[...] = new.astype(jnp.bfloat16)
            cnt_ref[0] = jnp.sum(new.astype(jnp.int32))

        return jnp.logical_or(done, cnt_ref[0] == 0)

    lax.fori_loop(0, MAX_ITERS, body, False)


_bfs = pl.pallas_call(
    _bfs_body,
    out_shape=jax.ShapeDtypeStruct((N, N), jnp.int32),
    scratch_shapes=[
        pltpu.VMEM((N, N), jnp.bfloat16),
        pltpu.VMEM((N, N), jnp.bfloat16),
        pltpu.SMEM((1,), jnp.int32),
    ],
)


@functools.partial(
    pl.kernel,
    out_type=jax.ShapeDtypeStruct((HEADS, N, N), jnp.float32),
    mesh=_mesh,
    scratch_types=[
        pltpu.VMEM((HEADS * TABLE_V,), jnp.float32),
        pltpu.VMEM((2, ROWS_PER_CHUNK, N), jnp.int32),
        pltpu.VMEM((2, HEADS, ROWS_PER_CHUNK, N), jnp.float32),
        pltpu.SemaphoreType.DMA((2,)),
        pltpu.SemaphoreType.DMA((2,)),
    ],
    compiler_params=_sc_params,
)
def _lookup(dist_hbm, ftab_hbm, out_hbm, tab_v, idx_v, out_v, in_sems, out_sems):
    wid = lax.axis_index("s") * NC + lax.axis_index("c")
    row_base = wid * ROWS_PER_TILE
    pltpu.sync_copy(ftab_hbm, tab_v)

    def in_desc(c, b):
        return pltpu.make_async_copy(
            dist_hbm.at[pl.ds(row_base + c * ROWS_PER_CHUNK, ROWS_PER_CHUNK), :],
            idx_v.at[b],
            in_sems.at[b],
        )

    def out_desc(c, b):
        return pltpu.make_async_copy(
            out_v.at[b],
            out_hbm.at[:, pl.ds(row_base + c * ROWS_PER_CHUNK, ROWS_PER_CHUNK), :],
            out_sems.at[b],
        )

    in_desc(0, 0).start()
    in_desc(1, 1).start()

    def chunk_pair(cp, carry):
        for b in range(2):
            c = cp * 2 + b
            in_desc(c, b).wait()

            @pl.when(cp > 0)
            def _():
                out_desc(c, b).wait()

            for r in range(ROWS_PER_CHUNK):
                @plsc.parallel_loop(0, N // LANES, 1, unroll=4)
                def _(cg):
                    col = cg * LANES
                    iv = idx_v[b, r, pl.ds(col, LANES)]
                    for h in range(HEADS):
                        vals = plsc.load_gather(tab_v, [iv + h * TABLE_V])
                        out_v[b, h, r, pl.ds(col, LANES)] = vals

            out_desc(c, b).start()

            @pl.when(c + 2 < CHUNKS)
            def _():
                in_desc(c + 2, b).start()
        return carry

    lax.fori_loop(0, CHUNKS // 2, chunk_pair, 0)
    out_desc(CHUNKS - 2, 0).wait()
    out_desc(CHUNKS - 1, 1).wait()


def kernel(x, edge_idx, table):
    del x
    a = _build_adj(edge_idx)
    dist = _bfs(a)
    ftab = jnp.transpose(table).reshape(HEADS * TABLE_V)
    return _lookup(dist, ftab)
